# baseline (speedup 1.0000x reference)
import functools

import jax
import jax.numpy as jnp
from jax import lax
from jax.experimental import pallas as pl
from jax.experimental.pallas import tpu as pltpu
from jax.experimental.pallas import tpu_sc as plsc

_T = 64


def kernel(inputs, pos_table):
    B, S, D = inputs.shape
    NC, NS = 2, 16
    NW = NC * NS
    seq_per_w = S // NW
    tiles = seq_per_w // _T
    mesh = plsc.VectorSubcoreMesh(core_axis_name="c", subcore_axis_name="s")

    @functools.partial(
        pl.kernel,
        mesh=mesh,
        out_type=jax.ShapeDtypeStruct((B, S, D), jnp.float32),
        scratch_types=[
            pltpu.VMEM((_T, D), jnp.float32),
            pltpu.VMEM((_T, D), jnp.float32),
        ],
    )
    def k(x_hbm, t_hbm, o_hbm, t_v, x_v):
        wid = lax.axis_index("s") * NC + lax.axis_index("c")
        base = wid * seq_per_w

        def tile_body(t, _):
            s0 = base + t * _T
            pltpu.sync_copy(t_hbm.at[pl.ds(s0, _T)], t_v)
            for b in range(B):
                pltpu.sync_copy(x_hbm.at[b, pl.ds(s0, _T)], x_v)

                def row_body(r, _):
                    for j in range(D // 16):
                        sl = pl.ds(j * 16, 16)
                        x_v[r, sl] = x_v[r, sl] + t_v[r, sl]
                    return 0

                lax.fori_loop(0, _T, row_body, 0)
                pltpu.sync_copy(x_v, o_hbm.at[b, pl.ds(s0, _T)])
            return 0

        lax.fori_loop(0, tiles, tile_body, 0)

    return k(inputs, pos_table)
